# Initial kernel scaffold; baseline (speedup 1.0000x reference)
import jax
import jax.numpy as jnp
from jax.experimental import pallas as pl


def kernel(x, edge_index, edge_attr, W1, b1, W2, b2, We1, be1, We2, be2, Wn1, bn1, Wn2, bn2):
    raise NotImplementedError("write your pallas kernel here")


# speedup vs baseline: 4.8446x; 4.8446x over previous
import functools

import jax
import jax.numpy as jnp
from jax import lax
from jax.experimental import pallas as pl
from jax.experimental.pallas import tpu as pltpu
from jax.experimental.pallas import tpu_sc as plsc

N = 10000
E = 160000
D = 256
HH = 128
NP = 10240
NBLK = 8
RB = NP // NBLK
NC = 2
NS = 16
RPT = NP // NS

EPD = E + 256
ETD = EPD // 32

ET = E // NS
LK = 200

EPE = 163840
ETE = EPE // 32
EK = 64

_PREC = jax.lax.Precision.HIGHEST


def _dot(a, b):
    return jnp.dot(a, b, preferred_element_type=jnp.float32, precision=_PREC)


def _mesh():
    return plsc.VectorSubcoreMesh(
        core_axis_name="c", subcore_axis_name="s",
        num_cores=NC, num_subcores=NS)


_SC_PARAMS = pltpu.CompilerParams(use_tc_tiling_on_sc=False)


@functools.cache
def _make_sc_deg():
    return functools.partial(
        pl.kernel,
        out_type=jax.ShapeDtypeStruct((NC, NP, 16), jnp.float32),
        mesh=_mesh(),
        scratch_types=[
            pltpu.VMEM((ETD,), jnp.int32),
            pltpu.VMEM((ETD, 16), jnp.float32),
            pltpu.VMEM_SHARED((NP, 16), jnp.float32),
        ],
        compiler_params=_SC_PARAMS,
    )(_sc_deg_body)


def _sc_deg_body(dstp_hbm, ones_hbm, zeros_hbm, out_hbm, idx_v, ones_v, deg_sp):
    cid = lax.axis_index("c")
    sid = lax.axis_index("s")
    w = sid * NC + cid
    stripe = pl.ds(pl.multiple_of(sid * RPT, 8), RPT)
    pltpu.sync_copy(zeros_hbm.at[stripe], deg_sp.at[stripe])
    pltpu.sync_copy(ones_hbm, ones_v)
    pltpu.sync_copy(dstp_hbm.at[pl.ds(pl.multiple_of(w * ETD, 8), ETD)], idx_v)
    plsc.subcore_barrier()
    pltpu.sync_copy(ones_v, deg_sp.at[idx_v], add=True)
    plsc.subcore_barrier()
    pltpu.sync_copy(deg_sp.at[stripe], out_hbm.at[cid, stripe])


@functools.cache
def _make_sc_layer():
    return functools.partial(
        pl.kernel,
        out_type=jax.ShapeDtypeStruct((2 * NP, HH), jnp.float32),
        mesh=_mesh(),
        scratch_types=[
            pltpu.VMEM((LK,), jnp.int32),
            pltpu.VMEM((LK,), jnp.int32),
            pltpu.VMEM((LK, HH), jnp.float32),
            pltpu.VMEM_SHARED((NP, HH), jnp.float32),
            pltpu.SemaphoreType.DMA,
        ],
        compiler_params=_SC_PARAMS,
    )(_sc_layer_body)


def _sc_layer_body(ycat_hbm, src2_hbm, dst_hbm, zcat_hbm,
                   src_v, dst_v, rows_v, z_sp, sem):
    cid = lax.axis_index("c")
    sid = lax.axis_index("s")
    stripe = pl.ds(pl.multiple_of(sid * RPT, 8), RPT)
    gstripe = pl.ds(pl.multiple_of(cid * NP + sid * RPT, 8), RPT)
    pltpu.sync_copy(ycat_hbm.at[gstripe], z_sp.at[stripe])
    plsc.subcore_barrier()

    def chunk(c, _):
        off = pl.multiple_of(sid * ET + c * LK, 8)
        pltpu.sync_copy(src2_hbm.at[cid, pl.ds(off, LK)], src_v)
        pltpu.sync_copy(dst_hbm.at[pl.ds(off, LK)], dst_v)
        pltpu.async_copy(ycat_hbm.at[src_v], rows_v, sem).wait()
        pltpu.sync_copy(rows_v, z_sp.at[dst_v], add=True)
        return 0

    lax.fori_loop(0, ET // LK, chunk, 0)
    plsc.subcore_barrier()
    pltpu.sync_copy(z_sp.at[stripe], zcat_hbm.at[gstripe])


@functools.cache
def _make_sc_edge():
    return functools.partial(
        pl.kernel,
        out_type=jax.ShapeDtypeStruct((EPE, 16), jnp.float32),
        mesh=_mesh(),
        scratch_types=[
            pltpu.VMEM((EK,), jnp.int32),
            pltpu.VMEM((EK,), jnp.int32),
            pltpu.VMEM((EK, D), jnp.float32),
            pltpu.VMEM((EK, D), jnp.float32),
            pltpu.VMEM((EK, D), jnp.float32),
            pltpu.VMEM((EK, 16), jnp.float32),
            pltpu.VMEM((D,), jnp.float32),
            pltpu.SemaphoreType.DMA,
        ],
        compiler_params=_SC_PARAMS,
    )(_sc_edge_body)


def _sc_edge_body(ab_hbm, esrc_hbm, edst_hbm, c_hbm, w2_hbm, out_hbm,
                  src_v, dst_v, a_v, b_v, c_v, o_v, w2_v, sem):
    cid = lax.axis_index("c")
    sid = lax.axis_index("s")
    w = sid * NC + cid
    pltpu.sync_copy(w2_hbm, w2_v)
    w2regs = [w2_v[pl.ds(k * 16, 16)] for k in range(16)]

    def chunk(c, _):
        off = pl.multiple_of(w * ETE + c * EK, 8)
        pltpu.sync_copy(esrc_hbm.at[pl.ds(off, EK)], src_v)
        pltpu.sync_copy(edst_hbm.at[pl.ds(off, EK)], dst_v)
        d1 = pltpu.async_copy(ab_hbm.at[src_v], a_v, sem)
        d2 = pltpu.async_copy(ab_hbm.at[dst_v], b_v, sem)
        d1.wait()
        d2.wait()
        pltpu.sync_copy(c_hbm.at[pl.ds(off, EK)], c_v)

        def edge(e, _):
            acc = jnp.zeros((16,), jnp.float32)
            for k in range(16):
                sl = pl.ds(k * 16, 16)
                v = a_v[e, sl] + b_v[e, sl] + c_v[e, sl]
                acc = acc + jnp.maximum(v, 0.0) * w2regs[k]
            o_v[e, :] = acc
            return 0

        lax.fori_loop(0, EK, edge, 0)
        pltpu.sync_copy(o_v, out_hbm.at[pl.ds(off, EK)])
        return 0

    lax.fori_loop(0, ETE // EK, chunk, 0)


def _run_deg(dstp, ones, zeros):
    return _make_sc_deg()(dstp, ones, zeros)


def _run_layer(ycat, src2, dst):
    return _make_sc_layer()(ycat, src2, dst)


def _run_edge(ab, esrc, edst, cmat, w2v):
    return _make_sc_edge()(ab, esrc, edst, cmat, w2v)


def _tc1_body(x_ref, w_ref, p_ref, y_ref, dinv_ref):
    deg = 1.0 + jnp.sum(p_ref[...], axis=(0, 2))
    dinv = lax.rsqrt(deg).reshape(RB, 1)
    xw = _dot(x_ref[...], w_ref[...])
    y_ref[...] = xw * dinv
    dinv_ref[...] = dinv


def _tc1(x_pad, W1, partials):
    return pl.pallas_call(
        _tc1_body,
        grid=(2, NBLK),
        in_specs=[
            pl.BlockSpec((RB, D), lambda j, i: (i, 0)),
            pl.BlockSpec((D, HH), lambda j, i: (0, j)),
            pl.BlockSpec((NC, RB, 16), lambda j, i: (0, i, 0)),
        ],
        out_specs=[
            pl.BlockSpec((RB, HH), lambda j, i: (j * NBLK + i, 0)),
            pl.BlockSpec((RB, 1), lambda j, i: (i, 0)),
        ],
        out_shape=[
            jax.ShapeDtypeStruct((2 * NP, HH), jnp.float32),
            jax.ShapeDtypeStruct((NP, 1), jnp.float32),
        ],
    )(x_pad, W1, partials)


def _tc2_body(za_ref, zb_ref, dinv_ref, b_ref, w_ref, y_ref):
    dinv = dinv_ref[...]
    z = jnp.concatenate([za_ref[...], zb_ref[...]], axis=1)
    h = jnp.maximum(dinv * z + b_ref[...], 0.0)
    y_ref[...] = _dot(h, w_ref[...]) * dinv


def _tc2(zcat, dinv, b1r, W2):
    return pl.pallas_call(
        _tc2_body,
        grid=(2, NBLK),
        in_specs=[
            pl.BlockSpec((RB, HH), lambda j, i: (i, 0)),
            pl.BlockSpec((RB, HH), lambda j, i: (NBLK + i, 0)),
            pl.BlockSpec((RB, 1), lambda j, i: (i, 0)),
            pl.BlockSpec((1, D), lambda j, i: (0, 0)),
            pl.BlockSpec((D, HH), lambda j, i: (0, j)),
        ],
        out_specs=pl.BlockSpec((RB, HH), lambda j, i: (j * NBLK + i, 0)),
        out_shape=jax.ShapeDtypeStruct((2 * NP, HH), jnp.float32),
    )(zcat, zcat, dinv, b1r, W2)


def _tc3_body(za_ref, zb_ref, dinv_ref, b_ref, we_ref, wn1_ref, bn1_ref,
              wn2_ref, bn2_ref, ab_ref, h_ref, n_ref):
    dinv = dinv_ref[...]
    z = jnp.concatenate([za_ref[...], zb_ref[...]], axis=1)
    h = jnp.maximum(dinv * z + b_ref[...], 0.0)
    h_ref[...] = h
    ab_ref[...] = _dot(h, we_ref[0])
    nmid = jnp.maximum(_dot(h, wn1_ref[...]) + bn1_ref[...], 0.0)
    n_ref[...] = _dot(nmid, wn2_ref[...]) + bn2_ref[...]


def _tc3(zcat, dinv, b2r, We1ab, Wn1, bn1r, Wn2, bn2r):
    return pl.pallas_call(
        _tc3_body,
        grid=(2, NBLK),
        in_specs=[
            pl.BlockSpec((RB, HH), lambda j, i: (i, 0)),
            pl.BlockSpec((RB, HH), lambda j, i: (NBLK + i, 0)),
            pl.BlockSpec((RB, 1), lambda j, i: (i, 0)),
            pl.BlockSpec((1, D), lambda j, i: (0, 0)),
            pl.BlockSpec((1, D, D), lambda j, i: (j, 0, 0)),
            pl.BlockSpec((D, HH), lambda j, i: (0, 0)),
            pl.BlockSpec((1, HH), lambda j, i: (0, 0)),
            pl.BlockSpec((HH, 1), lambda j, i: (0, 0)),
            pl.BlockSpec((1, 1), lambda j, i: (0, 0)),
        ],
        out_specs=[
            pl.BlockSpec((RB, D), lambda j, i: (j * NBLK + i, 0)),
            pl.BlockSpec((RB, D), lambda j, i: (i, 0)),
            pl.BlockSpec((RB, 1), lambda j, i: (i, 0)),
        ],
        out_shape=[
            jax.ShapeDtypeStruct((2 * NP, D), jnp.float32),
            jax.ShapeDtypeStruct((NP, D), jnp.float32),
            jax.ShapeDtypeStruct((NP, 1), jnp.float32),
        ],
    )(zcat, zcat, dinv, b2r, We1ab, Wn1, bn1r, Wn2, bn2r)


def _tc3b_body(ea_ref, we_ref, be_ref, c_ref):
    c_ref[...] = _dot(ea_ref[...], we_ref[...]) + be_ref[...]


def _tc3b(ea_pad, We1e, be1r):
    eb = 2048
    return pl.pallas_call(
        _tc3b_body,
        grid=(EPE // eb,),
        in_specs=[
            pl.BlockSpec((eb, 16), lambda i: (i, 0)),
            pl.BlockSpec((16, D), lambda i: (0, 0)),
            pl.BlockSpec((1, D), lambda i: (0, 0)),
        ],
        out_specs=pl.BlockSpec((eb, D), lambda i: (i, 0)),
        out_shape=jax.ShapeDtypeStruct((EPE, D), jnp.float32),
    )(ea_pad, We1e, be1r)


def _tc4_body(ep_ref, be2_ref, out_ref):
    out_ref[...] = jnp.sum(ep_ref[...], axis=1, keepdims=True) + be2_ref[...]


def _tc4(epart, be2r):
    eb = 4096
    return pl.pallas_call(
        _tc4_body,
        grid=(EPE // eb,),
        in_specs=[
            pl.BlockSpec((eb, 16), lambda i: (i, 0)),
            pl.BlockSpec((1, 1), lambda i: (0, 0)),
        ],
        out_specs=pl.BlockSpec((eb, 1), lambda i: (i, 0)),
        out_shape=jax.ShapeDtypeStruct((EPE, 1), jnp.float32),
    )(epart, be2r)


def kernel(x, edge_index, edge_attr, W1, b1, W2, b2, We1, be1, We2, be2,
           Wn1, bn1, Wn2, bn2):
    src = edge_index[0]
    dst = edge_index[1]

    dstp = jnp.concatenate(
        [dst, jnp.full((EPD - E,), NP - 1, jnp.int32)])
    deg_ones = jnp.full((ETD, 16), 0.0625, jnp.float32)
    deg_zeros = jnp.zeros((NP, 16), jnp.float32)
    src2 = jnp.stack([src, src + NP])
    pad_idx = (jnp.arange(EPE - E, dtype=jnp.int32) % N)
    esrc = jnp.concatenate([src, pad_idx])
    edst = jnp.concatenate([dst + NP, pad_idx + NP])
    x_pad = jnp.pad(x, ((0, NP - N), (0, 0)))
    ea_pad = jnp.pad(edge_attr, ((0, EPE - E), (0, 0)))

    b1r = b1.reshape(1, D)
    b2r = b2.reshape(1, D)
    be1r = be1.reshape(1, D)
    be2r = be2.reshape(1, 1)
    bn1r = bn1.reshape(1, HH)
    bn2r = bn2.reshape(1, 1)
    We1ab = jnp.stack([We1[:D], We1[D:2 * D]])
    We1e = We1[2 * D:]
    w2v = We2[:, 0]

    partials = _run_deg(dstp, deg_ones, deg_zeros)
    ycat, dinv = _tc1(x_pad, W1, partials)
    zcat = _run_layer(ycat, src2, dst)
    y2cat = _tc2(zcat, dinv, b1r, W2)
    z2cat = _run_layer(y2cat, src2, dst)
    abflat, h_pad, nout = _tc3(z2cat, dinv, b2r, We1ab, Wn1, bn1r, Wn2, bn2r)
    cmat = _tc3b(ea_pad, We1e, be1r)
    epart = _run_edge(abflat, esrc, edst, cmat, w2v)
    eout = _tc4(epart, be2r)

    return (eout[:E, 0], nout[:N, 0], h_pad[:N])


# speedup vs baseline: 5.5324x; 1.1420x over previous
import functools

import jax
import jax.numpy as jnp
from jax import lax
from jax.experimental import pallas as pl
from jax.experimental.pallas import tpu as pltpu
from jax.experimental.pallas import tpu_sc as plsc

N = 10000
E = 160000
D = 256
HH = 128
NP = 10240
NBLK = 8
RB = NP // NBLK
NC = 2
NS = 16
RPT = NP // NS

EPD = E + 256
ETD = EPD // 32

EPE = 163840
ETL = EPE // NS
LK = 128
NCHL = ETL // LK

ETE = EPE // NS
EK = 128
NCHE = ETE // EK

_PREC = jax.lax.Precision.HIGHEST


def _dot(a, b):
    return jnp.dot(a, b, preferred_element_type=jnp.float32, precision=_PREC)


def _mesh():
    return plsc.VectorSubcoreMesh(
        core_axis_name="c", subcore_axis_name="s",
        num_cores=NC, num_subcores=NS)


_SC_PARAMS = pltpu.CompilerParams(use_tc_tiling_on_sc=False)


@functools.cache
def _make_sc_deg():
    return functools.partial(
        pl.kernel,
        out_type=jax.ShapeDtypeStruct((NC, NP, 16), jnp.float32),
        mesh=_mesh(),
        scratch_types=[
            pltpu.VMEM((ETD,), jnp.int32),
            pltpu.VMEM((ETD, 16), jnp.float32),
            pltpu.VMEM_SHARED((NP, 16), jnp.float32),
        ],
        compiler_params=_SC_PARAMS,
    )(_sc_deg_body)


def _sc_deg_body(dstp_hbm, ones_hbm, zeros_hbm, out_hbm, idx_v, ones_v, deg_sp):
    cid = lax.axis_index("c")
    sid = lax.axis_index("s")
    w = sid * NC + cid
    stripe = pl.ds(pl.multiple_of(sid * RPT, 8), RPT)
    pltpu.sync_copy(zeros_hbm.at[stripe], deg_sp.at[stripe])
    pltpu.sync_copy(ones_hbm, ones_v)
    pltpu.sync_copy(dstp_hbm.at[pl.ds(pl.multiple_of(w * ETD, 8), ETD)], idx_v)
    plsc.subcore_barrier()
    pltpu.sync_copy(ones_v, deg_sp.at[idx_v], add=True)
    plsc.subcore_barrier()
    pltpu.sync_copy(deg_sp.at[stripe], out_hbm.at[cid, stripe])


@functools.cache
def _make_sc_layer():
    buf = lambda: [pltpu.VMEM((LK,), jnp.int32),
                   pltpu.VMEM((LK,), jnp.int32),
                   pltpu.VMEM((LK, HH), jnp.float32),
                   pltpu.SemaphoreType.DMA]
    return functools.partial(
        pl.kernel,
        out_type=jax.ShapeDtypeStruct((2 * NP, HH), jnp.float32),
        mesh=_mesh(),
        scratch_types=buf() + buf() + [
            pltpu.VMEM_SHARED((NP, HH), jnp.float32),
        ],
        compiler_params=_SC_PARAMS,
    )(_sc_layer_body)


def _sc_layer_body(ycat_hbm, src2_hbm, dst_hbm, zcat_hbm,
                   src0, dst0, rows0, sem0, src1, dst1, rows1, sem1, z_sp):
    cid = lax.axis_index("c")
    sid = lax.axis_index("s")
    bufs = ((src0, dst0, rows0, sem0), (src1, dst1, rows1, sem1))
    stripe = pl.ds(pl.multiple_of(sid * RPT, 8), RPT)
    gstripe = pl.ds(pl.multiple_of(cid * NP + sid * RPT, 8), RPT)
    pltpu.sync_copy(ycat_hbm.at[gstripe], z_sp.at[stripe])
    plsc.subcore_barrier()

    def off_of(c):
        return pl.multiple_of(sid * ETL + c * LK, 8)

    def prefetch(c, b):
        src_v, _, rows_v, sem = bufs[b]
        pltpu.sync_copy(src2_hbm.at[cid, pl.ds(off_of(c), LK)], src_v)
        pltpu.async_copy(ycat_hbm.at[src_v], rows_v, sem)

    def consume(c, b):
        src_v, dst_v, rows_v, sem = bufs[b]
        pltpu.sync_copy(dst_hbm.at[pl.ds(off_of(c), LK)], dst_v)
        pltpu.make_async_copy(ycat_hbm.at[src_v], rows_v, sem).wait()
        pltpu.sync_copy(rows_v, z_sp.at[dst_v], add=True)

    prefetch(0, 0)

    def step(c2, _):
        c0 = c2 * 2
        prefetch(c0 + 1, 1)
        consume(c0, 0)

        @pl.when(c0 + 2 < NCHL)
        def _():
            prefetch(c0 + 2, 0)

        consume(c0 + 1, 1)
        return 0

    lax.fori_loop(0, NCHL // 2, step, 0)
    plsc.subcore_barrier()
    pltpu.sync_copy(z_sp.at[stripe], zcat_hbm.at[gstripe])


@functools.cache
def _make_sc_edge():
    buf = lambda: [pltpu.VMEM((EK,), jnp.int32),
                   pltpu.VMEM((EK,), jnp.int32),
                   pltpu.VMEM((EK, HH), jnp.float32),
                   pltpu.VMEM((EK, HH), jnp.float32),
                   pltpu.VMEM((EK, HH), jnp.float32),
                   pltpu.VMEM((EK, 16), jnp.float32),
                   pltpu.SemaphoreType.DMA]
    return functools.partial(
        pl.kernel,
        out_type=jax.ShapeDtypeStruct((NC, EPE, 16), jnp.float32),
        mesh=_mesh(),
        scratch_types=buf() + buf() + [pltpu.VMEM((HH,), jnp.float32)],
        compiler_params=_SC_PARAMS,
    )(_sc_edge_body)


def _sc_edge_body(tcat_hbm, esrc2_hbm, edst2_hbm, ch_hbm, w2h_hbm, out_hbm,
                  s0, d0, a0, b0, c0, o0, sem0,
                  s1, d1, a1, b1, c1, o1, sem1, w2_v):
    cid = lax.axis_index("c")
    sid = lax.axis_index("s")
    bufs = ((s0, d0, a0, b0, c0, o0, sem0), (s1, d1, a1, b1, c1, o1, sem1))
    pltpu.sync_copy(w2h_hbm.at[cid], w2_v)
    w2regs = [w2_v[pl.ds(k * 16, 16)] for k in range(HH // 16)]

    def off_of(c):
        return pl.multiple_of(sid * ETE + c * EK, 8)

    def prefetch(c, b):
        src_v, dst_v, a_v, b_v, c_v, _, sem = bufs[b]
        off = off_of(c)
        pltpu.sync_copy(esrc2_hbm.at[cid, pl.ds(off, EK)], src_v)
        pltpu.sync_copy(edst2_hbm.at[cid, pl.ds(off, EK)], dst_v)
        pltpu.async_copy(tcat_hbm.at[src_v], a_v, sem)
        pltpu.async_copy(tcat_hbm.at[dst_v], b_v, sem)
        pltpu.async_copy(ch_hbm.at[cid, pl.ds(off, EK)], c_v, sem)

    def consume(c, b):
        src_v, dst_v, a_v, b_v, c_v, o_v, sem = bufs[b]
        off = off_of(c)
        pltpu.make_async_copy(tcat_hbm.at[src_v], a_v, sem).wait()
        pltpu.make_async_copy(tcat_hbm.at[dst_v], b_v, sem).wait()
        pltpu.make_async_copy(ch_hbm.at[cid, pl.ds(off, EK)], c_v, sem).wait()

        def edge(e, _):
            acc = jnp.zeros((16,), jnp.float32)
            for k in range(HH // 16):
                sl = pl.ds(k * 16, 16)
                v = a_v[e, sl] + b_v[e, sl] + c_v[e, sl]
                acc = acc + jnp.maximum(v, 0.0) * w2regs[k]
            o_v[e, :] = acc
            return 0

        lax.fori_loop(0, EK, edge, 0)
        pltpu.sync_copy(o_v, out_hbm.at[cid, pl.ds(off, EK)])

    prefetch(0, 0)

    def step(c2, _):
        ch = c2 * 2
        prefetch(ch + 1, 1)
        consume(ch, 0)

        @pl.when(ch + 2 < NCHE)
        def _():
            prefetch(ch + 2, 0)

        consume(ch + 1, 1)
        return 0

    lax.fori_loop(0, NCHE // 2, step, 0)


def _run_deg(dstp, ones, zeros):
    return _make_sc_deg()(dstp, ones, zeros)


def _run_layer(ycat, src2, dst):
    return _make_sc_layer()(ycat, src2, dst)


def _run_edge(ab, esrc, edst, cmat, w2v):
    return _make_sc_edge()(ab, esrc, edst, cmat, w2v)


def _tc1_body(x_ref, w_ref, p_ref, y_ref, dinv_ref):
    deg = 1.0 + jnp.sum(p_ref[...], axis=(0, 2))
    dinv = lax.rsqrt(deg).reshape(RB, 1)
    xw = _dot(x_ref[...], w_ref[...])
    y_ref[...] = xw * dinv
    dinv_ref[...] = dinv


def _tc1(x_pad, W1, partials):
    return pl.pallas_call(
        _tc1_body,
        grid=(2, NBLK),
        in_specs=[
            pl.BlockSpec((RB, D), lambda j, i: (i, 0)),
            pl.BlockSpec((D, HH), lambda j, i: (0, j)),
            pl.BlockSpec((NC, RB, 16), lambda j, i: (0, i, 0)),
        ],
        out_specs=[
            pl.BlockSpec((RB, HH), lambda j, i: (j * NBLK + i, 0)),
            pl.BlockSpec((RB, 1), lambda j, i: (i, 0)),
        ],
        out_shape=[
            jax.ShapeDtypeStruct((2 * NP, HH), jnp.float32),
            jax.ShapeDtypeStruct((NP, 1), jnp.float32),
        ],
    )(x_pad, W1, partials)


def _tc2_body(za_ref, zb_ref, dinv_ref, b_ref, w_ref, y_ref):
    dinv = dinv_ref[...]
    z = jnp.concatenate([za_ref[...], zb_ref[...]], axis=1)
    h = jnp.maximum(dinv * z + b_ref[...], 0.0)
    y_ref[...] = _dot(h, w_ref[...]) * dinv


def _tc2(zcat, dinv, b1r, W2):
    return pl.pallas_call(
        _tc2_body,
        grid=(2, NBLK),
        in_specs=[
            pl.BlockSpec((RB, HH), lambda j, i: (i, 0)),
            pl.BlockSpec((RB, HH), lambda j, i: (NBLK + i, 0)),
            pl.BlockSpec((RB, 1), lambda j, i: (i, 0)),
            pl.BlockSpec((1, D), lambda j, i: (0, 0)),
            pl.BlockSpec((D, HH), lambda j, i: (0, j)),
        ],
        out_specs=pl.BlockSpec((RB, HH), lambda j, i: (j * NBLK + i, 0)),
        out_shape=jax.ShapeDtypeStruct((2 * NP, HH), jnp.float32),
    )(zcat, zcat, dinv, b1r, W2)


def _tc3_body(za_ref, zb_ref, dinv_ref, b_ref, we_ref, wn1_ref, bn1_ref,
              wn2_ref, bn2_ref, t_ref, h_ref, n_ref):
    dinv = dinv_ref[...]
    z = jnp.concatenate([za_ref[...], zb_ref[...]], axis=1)
    h = jnp.maximum(dinv * z + b_ref[...], 0.0)
    h_ref[...] = h
    t_ref[...] = _dot(h, we_ref[0])
    nmid = jnp.maximum(_dot(h, wn1_ref[...]) + bn1_ref[...], 0.0)
    n_ref[...] = _dot(nmid, wn2_ref[...]) + bn2_ref[...]


def _tc3(zcat, dinv, b2r, We1ab, Wn1, bn1r, Wn2, bn2r):
    return pl.pallas_call(
        _tc3_body,
        grid=(2, 2, NBLK),
        in_specs=[
            pl.BlockSpec((RB, HH), lambda j, p, i: (i, 0)),
            pl.BlockSpec((RB, HH), lambda j, p, i: (NBLK + i, 0)),
            pl.BlockSpec((RB, 1), lambda j, p, i: (i, 0)),
            pl.BlockSpec((1, D), lambda j, p, i: (0, 0)),
            pl.BlockSpec((1, D, HH), lambda j, p, i: (j, 0, p)),
            pl.BlockSpec((D, HH), lambda j, p, i: (0, 0)),
            pl.BlockSpec((1, HH), lambda j, p, i: (0, 0)),
            pl.BlockSpec((HH, 1), lambda j, p, i: (0, 0)),
            pl.BlockSpec((1, 1), lambda j, p, i: (0, 0)),
        ],
        out_specs=[
            pl.BlockSpec((RB, HH), lambda j, p, i: ((p * 2 + j) * NBLK + i, 0)),
            pl.BlockSpec((RB, D), lambda j, p, i: (i, 0)),
            pl.BlockSpec((RB, 1), lambda j, p, i: (i, 0)),
        ],
        out_shape=[
            jax.ShapeDtypeStruct((4 * NP, HH), jnp.float32),
            jax.ShapeDtypeStruct((NP, D), jnp.float32),
            jax.ShapeDtypeStruct((NP, 1), jnp.float32),
        ],
    )(zcat, zcat, dinv, b2r, We1ab, Wn1, bn1r, Wn2, bn2r)


def _tc3b_body(ea_ref, we_ref, be_ref, c_ref):
    c_ref[0] = _dot(ea_ref[0], we_ref[...]) + be_ref[...]


def _tc3b(ea_pad3, We1e, be1r):
    eb = 2048
    return pl.pallas_call(
        _tc3b_body,
        grid=(2, EPE // eb),
        in_specs=[
            pl.BlockSpec((1, eb, 16), lambda j, i: (0, i, 0)),
            pl.BlockSpec((16, HH), lambda j, i: (0, j)),
            pl.BlockSpec((1, HH), lambda j, i: (0, j)),
        ],
        out_specs=pl.BlockSpec((1, eb, HH), lambda j, i: (j, i, 0)),
        out_shape=jax.ShapeDtypeStruct((NC, EPE, HH), jnp.float32),
    )(ea_pad3, We1e, be1r)


def _tc4_body(ep_ref, be2_ref, out_ref):
    out_ref[...] = jnp.sum(ep_ref[...], axis=(0, 2))[:, None] + be2_ref[...]


def _tc4(epart, be2r):
    eb = 4096
    return pl.pallas_call(
        _tc4_body,
        grid=(EPE // eb,),
        in_specs=[
            pl.BlockSpec((NC, eb, 16), lambda i: (0, i, 0)),
            pl.BlockSpec((1, 1), lambda i: (0, 0)),
        ],
        out_specs=pl.BlockSpec((eb, 1), lambda i: (i, 0)),
        out_shape=jax.ShapeDtypeStruct((EPE, 1), jnp.float32),
    )(epart, be2r)


def kernel(x, edge_index, edge_attr, W1, b1, W2, b2, We1, be1, We2, be2,
           Wn1, bn1, Wn2, bn2):
    src = edge_index[0]
    dst = edge_index[1]

    dstp = jnp.concatenate(
        [dst, jnp.full((EPD - E,), NP - 1, jnp.int32)])
    deg_ones = jnp.full((ETD, 16), 0.0625, jnp.float32)
    deg_zeros = jnp.zeros((NP, 16), jnp.float32)
    pad_idx = (jnp.arange(EPE - E, dtype=jnp.int32) % N)
    srcp = jnp.concatenate([src, pad_idx])
    dstl = jnp.concatenate(
        [dst, N + (jnp.arange(EPE - E, dtype=jnp.int32) % (NP - N))])
    src2 = jnp.stack([srcp, srcp + NP])
    dstp_e = jnp.concatenate([dst, pad_idx])
    esrc2 = jnp.stack([srcp, srcp + 2 * NP])
    edst2 = jnp.stack([dstp_e + NP, dstp_e + 3 * NP])
    x_pad = jnp.pad(x, ((0, NP - N), (0, 0)))
    ea_pad3 = jnp.pad(edge_attr, ((0, EPE - E), (0, 0)))[None]

    b1r = b1.reshape(1, D)
    b2r = b2.reshape(1, D)
    be1r = be1.reshape(1, D)
    be2r = be2.reshape(1, 1)
    bn1r = bn1.reshape(1, HH)
    bn2r = bn2.reshape(1, 1)
    We1ab = jnp.stack([We1[:D], We1[D:2 * D]])
    We1e = We1[2 * D:]
    w2h = We2[:, 0].reshape(NC, HH)

    partials = _run_deg(dstp, deg_ones, deg_zeros)
    ycat, dinv = _tc1(x_pad, W1, partials)
    zcat = _run_layer(ycat, src2, dstl)
    y2cat = _tc2(zcat, dinv, b1r, W2)
    z2cat = _run_layer(y2cat, src2, dstl)
    tcat, h_pad, nout = _tc3(z2cat, dinv, b2r, We1ab, Wn1, bn1r, Wn2, bn2r)
    ch = _tc3b(ea_pad3, We1e, be1r)
    epart = _run_edge(tcat, esrc2, edst2, ch, w2h)
    eout = _tc4(epart, be2r)

    return (eout[:E, 0], nout[:N, 0], h_pad[:N])


# speedup vs baseline: 6.9080x; 1.2487x over previous
import functools

import jax
import jax.numpy as jnp
from jax import lax
from jax.experimental import pallas as pl
from jax.experimental.pallas import tpu as pltpu
from jax.experimental.pallas import tpu_sc as plsc

N = 10000
E = 160000
D = 256
HH = 128
NP = 10240
NBLK = 8
RB = NP // NBLK
NC = 2
NS = 16
RPT = NP // NS

EPD = E + 256
ETD = EPD // 32

EPE = 163840
ETL = EPE // NS
LK = 128
NCHL = ETL // LK

ETE = EPE // NS
EK = 128
NCHE = ETE // EK

def _dot(a, b, prec=jax.lax.Precision.DEFAULT):
    return jnp.dot(a, b, preferred_element_type=jnp.float32, precision=prec)


def _mesh():
    return plsc.VectorSubcoreMesh(
        core_axis_name="c", subcore_axis_name="s",
        num_cores=NC, num_subcores=NS)


_SC_PARAMS = pltpu.CompilerParams(use_tc_tiling_on_sc=False)


@functools.cache
def _make_sc_deg():
    return functools.partial(
        pl.kernel,
        out_type=jax.ShapeDtypeStruct((NC, NP, 16), jnp.float32),
        mesh=_mesh(),
        scratch_types=[
            pltpu.VMEM((ETD,), jnp.int32),
            pltpu.VMEM((ETD, 16), jnp.float32),
            pltpu.VMEM_SHARED((NP, 16), jnp.float32),
        ],
        compiler_params=_SC_PARAMS,
    )(_sc_deg_body)


def _sc_deg_body(dstp_hbm, ones_hbm, zeros_hbm, out_hbm, idx_v, ones_v, deg_sp):
    cid = lax.axis_index("c")
    sid = lax.axis_index("s")
    w = sid * NC + cid
    stripe = pl.ds(pl.multiple_of(sid * RPT, 8), RPT)
    pltpu.sync_copy(zeros_hbm.at[stripe], deg_sp.at[stripe])
    pltpu.sync_copy(ones_hbm, ones_v)
    pltpu.sync_copy(dstp_hbm.at[pl.ds(pl.multiple_of(w * ETD, 8), ETD)], idx_v)
    plsc.subcore_barrier()
    pltpu.sync_copy(ones_v, deg_sp.at[idx_v], add=True)
    plsc.subcore_barrier()
    pltpu.sync_copy(deg_sp.at[stripe], out_hbm.at[cid, stripe])


@functools.cache
def _make_sc_layer():
    buf = lambda: [pltpu.VMEM((LK,), jnp.int32),
                   pltpu.VMEM((LK,), jnp.int32),
                   pltpu.VMEM((LK, HH), jnp.float32),
                   pltpu.SemaphoreType.DMA]
    return functools.partial(
        pl.kernel,
        out_type=jax.ShapeDtypeStruct((2 * NP, HH), jnp.float32),
        mesh=_mesh(),
        scratch_types=buf() + buf() + [
            pltpu.VMEM_SHARED((NP, HH), jnp.float32),
        ],
        compiler_params=_SC_PARAMS,
    )(_sc_layer_body)


def _sc_layer_body(ycat_hbm, src2_hbm, dst_hbm, zcat_hbm,
                   src0, dst0, rows0, sem0, src1, dst1, rows1, sem1, z_sp):
    cid = lax.axis_index("c")
    sid = lax.axis_index("s")
    bufs = ((src0, dst0, rows0, sem0), (src1, dst1, rows1, sem1))
    stripe = pl.ds(pl.multiple_of(sid * RPT, 8), RPT)
    gstripe = pl.ds(pl.multiple_of(cid * NP + sid * RPT, 8), RPT)
    pltpu.sync_copy(ycat_hbm.at[gstripe], z_sp.at[stripe])
    plsc.subcore_barrier()

    def off_of(c):
        return pl.multiple_of(sid * ETL + c * LK, 8)

    def prefetch(c, b):
        src_v, _, rows_v, sem = bufs[b]
        pltpu.sync_copy(src2_hbm.at[cid, pl.ds(off_of(c), LK)], src_v)
        pltpu.async_copy(ycat_hbm.at[src_v], rows_v, sem)

    def consume(c, b):
        src_v, dst_v, rows_v, sem = bufs[b]
        pltpu.sync_copy(dst_hbm.at[pl.ds(off_of(c), LK)], dst_v)
        pltpu.make_async_copy(ycat_hbm.at[src_v], rows_v, sem).wait()
        pltpu.sync_copy(rows_v, z_sp.at[dst_v], add=True)

    prefetch(0, 0)

    def step(c2, _):
        c0 = c2 * 2
        prefetch(c0 + 1, 1)
        consume(c0, 0)

        @pl.when(c0 + 2 < NCHL)
        def _():
            prefetch(c0 + 2, 0)

        consume(c0 + 1, 1)
        return 0

    lax.fori_loop(0, NCHL // 2, step, 0)
    plsc.subcore_barrier()
    pltpu.sync_copy(z_sp.at[stripe], zcat_hbm.at[gstripe])


@functools.cache
def _make_sc_edge():
    buf = lambda: [pltpu.VMEM((EK,), jnp.int32),
                   pltpu.VMEM((EK,), jnp.int32),
                   pltpu.VMEM((EK, HH), jnp.float32),
                   pltpu.VMEM((EK, HH), jnp.float32),
                   pltpu.VMEM((EK, HH), jnp.float32),
                   pltpu.VMEM((EK, 16), jnp.float32),
                   pltpu.SemaphoreType.DMA]
    return functools.partial(
        pl.kernel,
        out_type=jax.ShapeDtypeStruct((NC, EPE, 16), jnp.float32),
        mesh=_mesh(),
        scratch_types=buf() + buf() + [pltpu.VMEM((HH,), jnp.float32)],
        compiler_params=_SC_PARAMS,
    )(_sc_edge_body)


def _sc_edge_body(tcat_hbm, esrc2_hbm, edst2_hbm, ch_hbm, w2h_hbm, out_hbm,
                  s0, d0, a0, b0, c0, o0, sem0,
                  s1, d1, a1, b1, c1, o1, sem1, w2_v):
    cid = lax.axis_index("c")
    sid = lax.axis_index("s")
    bufs = ((s0, d0, a0, b0, c0, o0, sem0), (s1, d1, a1, b1, c1, o1, sem1))
    pltpu.sync_copy(w2h_hbm.at[cid], w2_v)
    w2regs = [w2_v[pl.ds(k * 16, 16)] for k in range(HH // 16)]

    def off_of(c):
        return pl.multiple_of(sid * ETE + c * EK, 8)

    def prefetch(c, b):
        src_v, dst_v, a_v, b_v, c_v, _, sem = bufs[b]
        off = off_of(c)
        pltpu.sync_copy(esrc2_hbm.at[cid, pl.ds(off, EK)], src_v)
        pltpu.sync_copy(edst2_hbm.at[cid, pl.ds(off, EK)], dst_v)
        pltpu.async_copy(tcat_hbm.at[src_v], a_v, sem)
        pltpu.async_copy(tcat_hbm.at[dst_v], b_v, sem)
        pltpu.async_copy(ch_hbm.at[cid, pl.ds(off, EK)], c_v, sem)

    def consume(c, b):
        src_v, dst_v, a_v, b_v, c_v, o_v, sem = bufs[b]
        off = off_of(c)
        pltpu.make_async_copy(tcat_hbm.at[src_v], a_v, sem).wait()
        pltpu.make_async_copy(tcat_hbm.at[dst_v], b_v, sem).wait()
        pltpu.make_async_copy(ch_hbm.at[cid, pl.ds(off, EK)], c_v, sem).wait()

        def edge(e, _):
            acc = jnp.zeros((16,), jnp.float32)
            for k in range(HH // 16):
                sl = pl.ds(k * 16, 16)
                v = a_v[e, sl] + b_v[e, sl] + c_v[e, sl]
                acc = acc + jnp.maximum(v, 0.0) * w2regs[k]
            o_v[e, :] = acc
            return 0

        lax.fori_loop(0, EK, edge, 0)
        pltpu.sync_copy(o_v, out_hbm.at[cid, pl.ds(off, EK)])

    prefetch(0, 0)

    def step(c2, _):
        ch = c2 * 2
        prefetch(ch + 1, 1)
        consume(ch, 0)

        @pl.when(ch + 2 < NCHE)
        def _():
            prefetch(ch + 2, 0)

        consume(ch + 1, 1)
        return 0

    lax.fori_loop(0, NCHE // 2, step, 0)


def _run_deg(dstp, ones, zeros):
    return _make_sc_deg()(dstp, ones, zeros)


def _run_layer(ycat, src2, dst):
    return _make_sc_layer()(ycat, src2, dst)


def _run_edge(ab, esrc, edst, cmat, w2v):
    return _make_sc_edge()(ab, esrc, edst, cmat, w2v)


def _tc1_body(x_ref, w_ref, p_ref, y_ref, dinv_ref):
    deg = 1.0 + jnp.sum(p_ref[...], axis=(0, 2))
    dinv = lax.rsqrt(deg).reshape(RB, 1)
    xw = _dot(x_ref[...], w_ref[...])
    y_ref[...] = xw * dinv
    dinv_ref[...] = dinv


def _tc1(x_pad, W1, partials):
    return pl.pallas_call(
        _tc1_body,
        grid=(2, NBLK),
        in_specs=[
            pl.BlockSpec((RB, D), lambda j, i: (i, 0)),
            pl.BlockSpec((D, HH), lambda j, i: (0, j)),
            pl.BlockSpec((NC, RB, 16), lambda j, i: (0, i, 0)),
        ],
        out_specs=[
            pl.BlockSpec((RB, HH), lambda j, i: (j * NBLK + i, 0)),
            pl.BlockSpec((RB, 1), lambda j, i: (i, 0)),
        ],
        out_shape=[
            jax.ShapeDtypeStruct((2 * NP, HH), jnp.float32),
            jax.ShapeDtypeStruct((NP, 1), jnp.float32),
        ],
    )(x_pad, W1, partials)


def _tc2_body(za_ref, zb_ref, dinv_ref, b_ref, w_ref, y_ref):
    dinv = dinv_ref[...]
    z = jnp.concatenate([za_ref[...], zb_ref[...]], axis=1)
    h = jnp.maximum(dinv * z + b_ref[...], 0.0)
    y_ref[...] = _dot(h, w_ref[...]) * dinv


def _tc2(zcat, dinv, b1r, W2):
    return pl.pallas_call(
        _tc2_body,
        grid=(2, NBLK),
        in_specs=[
            pl.BlockSpec((RB, HH), lambda j, i: (i, 0)),
            pl.BlockSpec((RB, HH), lambda j, i: (NBLK + i, 0)),
            pl.BlockSpec((RB, 1), lambda j, i: (i, 0)),
            pl.BlockSpec((1, D), lambda j, i: (0, 0)),
            pl.BlockSpec((D, HH), lambda j, i: (0, j)),
        ],
        out_specs=pl.BlockSpec((RB, HH), lambda j, i: (j * NBLK + i, 0)),
        out_shape=jax.ShapeDtypeStruct((2 * NP, HH), jnp.float32),
    )(zcat, zcat, dinv, b1r, W2)


def _tc3_body(za_ref, zb_ref, dinv_ref, b_ref, we_ref, wn1_ref, bn1_ref,
              wn2_ref, bn2_ref, t_ref, h_ref, n_ref):
    dinv = dinv_ref[...]
    z = jnp.concatenate([za_ref[...], zb_ref[...]], axis=1)
    h = jnp.maximum(dinv * z + b_ref[...], 0.0)
    h_ref[...] = h
    t_ref[...] = _dot(h, we_ref[0])
    nmid = jnp.maximum(_dot(h, wn1_ref[...]) + bn1_ref[...], 0.0)
    n_ref[...] = _dot(nmid, wn2_ref[...]) + bn2_ref[...]


def _tc3(zcat, dinv, b2r, We1ab, Wn1, bn1r, Wn2, bn2r):
    return pl.pallas_call(
        _tc3_body,
        grid=(2, 2, NBLK),
        in_specs=[
            pl.BlockSpec((RB, HH), lambda j, p, i: (i, 0)),
            pl.BlockSpec((RB, HH), lambda j, p, i: (NBLK + i, 0)),
            pl.BlockSpec((RB, 1), lambda j, p, i: (i, 0)),
            pl.BlockSpec((1, D), lambda j, p, i: (0, 0)),
            pl.BlockSpec((1, D, HH), lambda j, p, i: (j, 0, p)),
            pl.BlockSpec((D, HH), lambda j, p, i: (0, 0)),
            pl.BlockSpec((1, HH), lambda j, p, i: (0, 0)),
            pl.BlockSpec((HH, 1), lambda j, p, i: (0, 0)),
            pl.BlockSpec((1, 1), lambda j, p, i: (0, 0)),
        ],
        out_specs=[
            pl.BlockSpec((RB, HH), lambda j, p, i: ((p * 2 + j) * NBLK + i, 0)),
            pl.BlockSpec((RB, D), lambda j, p, i: (i, 0)),
            pl.BlockSpec((RB, 1), lambda j, p, i: (i, 0)),
        ],
        out_shape=[
            jax.ShapeDtypeStruct((4 * NP, HH), jnp.float32),
            jax.ShapeDtypeStruct((NP, D), jnp.float32),
            jax.ShapeDtypeStruct((NP, 1), jnp.float32),
        ],
    )(zcat, zcat, dinv, b2r, We1ab, Wn1, bn1r, Wn2, bn2r)


def _tc3b_body(ea_ref, we_ref, be_ref, c_ref):
    c_ref[0] = _dot(ea_ref[0], we_ref[...]) + be_ref[...]


def _tc3b(ea_pad3, We1e, be1r):
    eb = 2048
    return pl.pallas_call(
        _tc3b_body,
        grid=(2, EPE // eb),
        in_specs=[
            pl.BlockSpec((1, eb, 16), lambda j, i: (0, i, 0)),
            pl.BlockSpec((16, HH), lambda j, i: (0, j)),
            pl.BlockSpec((1, HH), lambda j, i: (0, j)),
        ],
        out_specs=pl.BlockSpec((1, eb, HH), lambda j, i: (j, i, 0)),
        out_shape=jax.ShapeDtypeStruct((NC, EPE, HH), jnp.float32),
    )(ea_pad3, We1e, be1r)


def _tc4_body(ep_ref, be2_ref, out_ref):
    v = ep_ref[0] + ep_ref[1]
    r = lax.broadcasted_iota(jnp.int32, (HH, 8), 0) // 16
    c = lax.broadcasted_iota(jnp.int32, (HH, 8), 1)
    mask = (r == c).astype(jnp.float32)
    out_ref[...] = _dot(v, mask, jax.lax.Precision.HIGHEST) + be2_ref[...]


def _tc4(epart8, be2r):
    eb8 = 512
    return pl.pallas_call(
        _tc4_body,
        grid=(EPE // 8 // eb8,),
        in_specs=[
            pl.BlockSpec((NC, eb8, HH), lambda i: (0, i, 0)),
            pl.BlockSpec((1, 1), lambda i: (0, 0)),
        ],
        out_specs=pl.BlockSpec((eb8, 8), lambda i: (i, 0)),
        out_shape=jax.ShapeDtypeStruct((EPE // 8, 8), jnp.float32),
    )(epart8, be2r)


def kernel(x, edge_index, edge_attr, W1, b1, W2, b2, We1, be1, We2, be2,
           Wn1, bn1, Wn2, bn2):
    src = edge_index[0]
    dst = edge_index[1]

    dstp = jnp.concatenate(
        [dst, jnp.full((EPD - E,), NP - 1, jnp.int32)])
    deg_ones = jnp.full((ETD, 16), 0.0625, jnp.float32)
    deg_zeros = jnp.zeros((NP, 16), jnp.float32)
    pad_idx = (jnp.arange(EPE - E, dtype=jnp.int32) % N)
    srcp = jnp.concatenate([src, pad_idx])
    dstl = jnp.concatenate(
        [dst, N + (jnp.arange(EPE - E, dtype=jnp.int32) % (NP - N))])
    src2 = jnp.stack([srcp, srcp + NP])
    dstp_e = jnp.concatenate([dst, pad_idx])
    esrc2 = jnp.stack([srcp, srcp + 2 * NP])
    edst2 = jnp.stack([dstp_e + NP, dstp_e + 3 * NP])
    x_pad = jnp.pad(x, ((0, NP - N), (0, 0)))
    ea8 = edge_attr.reshape(E * 16 // HH, HH)
    ea_pad3 = jnp.pad(ea8, ((0, (EPE - E) * 16 // HH, ), (0, 0))) \
        .reshape(EPE, 16)[None]

    b1r = b1.reshape(1, D)
    b2r = b2.reshape(1, D)
    be1r = be1.reshape(1, D)
    be2r = be2.reshape(1, 1)
    bn1r = bn1.reshape(1, HH)
    bn2r = bn2.reshape(1, 1)
    We1ab = jnp.stack([We1[:D], We1[D:2 * D]])
    We1e = We1[2 * D:]
    w2h = We2[:, 0].reshape(NC, HH)

    partials = _run_deg(dstp, deg_ones, deg_zeros)
    ch = _tc3b(ea_pad3, We1e, be1r)
    ycat, dinv = _tc1(x_pad, W1, partials)
    zcat = _run_layer(ycat, src2, dstl)
    y2cat = _tc2(zcat, dinv, b1r, W2)
    z2cat = _run_layer(y2cat, src2, dstl)
    tcat, h_pad, nout = _tc3(z2cat, dinv, b2r, We1ab, Wn1, bn1r, Wn2, bn2r)
    epart = _run_edge(tcat, esrc2, edst2, ch, w2h)
    epart8 = epart.reshape(NC, EPE * 16 // HH, HH)
    eout = _tc4(epart8, be2r)

    return (eout.reshape(EPE)[:E], nout[:N, 0], h_pad[:N])


# speedup vs baseline: 7.2314x; 1.0468x over previous
import functools

import jax
import jax.numpy as jnp
from jax import lax
from jax.experimental import pallas as pl
from jax.experimental.pallas import tpu as pltpu
from jax.experimental.pallas import tpu_sc as plsc

N = 10000
E = 160000
D = 256
HH = 128
NP = 10240
NBLK = 8
RB = NP // NBLK
NC = 2
NS = 16
RPT = NP // NS

EPD = E + 256
ETD = EPD // 32

EPE = 163840
ETL = EPE // NS
LK = 128
NCHL = ETL // LK

ETE = EPE // NS
EK = 128
NCHE = ETE // EK

def _dot(a, b, prec=jax.lax.Precision.DEFAULT):
    return jnp.dot(a, b, preferred_element_type=jnp.float32, precision=prec)


def _mesh():
    return plsc.VectorSubcoreMesh(
        core_axis_name="c", subcore_axis_name="s",
        num_cores=NC, num_subcores=NS)


_SC_PARAMS = pltpu.CompilerParams(use_tc_tiling_on_sc=False)


@functools.cache
def _make_sc_deg():
    return functools.partial(
        pl.kernel,
        out_type=jax.ShapeDtypeStruct((NC, NP, 16), jnp.float32),
        mesh=_mesh(),
        scratch_types=[
            pltpu.VMEM((ETD,), jnp.int32),
            pltpu.VMEM((ETD, 16), jnp.float32),
            pltpu.VMEM_SHARED((NP, 16), jnp.float32),
        ],
        compiler_params=_SC_PARAMS,
    )(_sc_deg_body)


def _sc_deg_body(dstp_hbm, ones_hbm, zeros_hbm, out_hbm, idx_v, ones_v, deg_sp):
    cid = lax.axis_index("c")
    sid = lax.axis_index("s")
    w = sid * NC + cid
    stripe = pl.ds(pl.multiple_of(sid * RPT, 8), RPT)
    pltpu.sync_copy(zeros_hbm.at[stripe], deg_sp.at[stripe])
    pltpu.sync_copy(ones_hbm, ones_v)
    pltpu.sync_copy(dstp_hbm.at[pl.ds(pl.multiple_of(w * ETD, 8), ETD)], idx_v)
    plsc.subcore_barrier()
    pltpu.sync_copy(ones_v, deg_sp.at[idx_v], add=True)
    plsc.subcore_barrier()
    pltpu.sync_copy(deg_sp.at[stripe], out_hbm.at[cid, stripe])


@functools.cache
def _make_sc_layer():
    buf = lambda: [pltpu.VMEM((LK,), jnp.int32),
                   pltpu.VMEM((LK,), jnp.int32),
                   pltpu.VMEM((LK, HH), jnp.float32),
                   pltpu.SemaphoreType.DMA]
    return functools.partial(
        pl.kernel,
        out_type=jax.ShapeDtypeStruct((2 * NP, HH), jnp.float32),
        mesh=_mesh(),
        scratch_types=buf() + buf() + [
            pltpu.VMEM_SHARED((NP, HH), jnp.float32),
        ],
        compiler_params=_SC_PARAMS,
    )(_sc_layer_body)


def _sc_layer_body(ycat_hbm, src2_hbm, dst_hbm, zcat_hbm,
                   src0, dst0, rows0, sem0, src1, dst1, rows1, sem1, z_sp):
    cid = lax.axis_index("c")
    sid = lax.axis_index("s")
    bufs = ((src0, dst0, rows0, sem0), (src1, dst1, rows1, sem1))
    stripe = pl.ds(pl.multiple_of(sid * RPT, 8), RPT)
    gstripe = pl.ds(pl.multiple_of(cid * NP + sid * RPT, 8), RPT)
    pltpu.sync_copy(ycat_hbm.at[gstripe], z_sp.at[stripe])
    plsc.subcore_barrier()

    def off_of(c):
        return pl.multiple_of(sid * ETL + c * LK, 8)

    def prefetch(c, b):
        src_v, _, rows_v, sem = bufs[b]
        pltpu.sync_copy(src2_hbm.at[cid, pl.ds(off_of(c), LK)], src_v)
        pltpu.async_copy(ycat_hbm.at[src_v], rows_v, sem)

    def consume(c, b):
        src_v, dst_v, rows_v, sem = bufs[b]
        pltpu.sync_copy(dst_hbm.at[pl.ds(off_of(c), LK)], dst_v)
        pltpu.make_async_copy(ycat_hbm.at[src_v], rows_v, sem).wait()
        pltpu.sync_copy(rows_v, z_sp.at[dst_v], add=True)

    prefetch(0, 0)

    def step(c2, _):
        c0 = c2 * 2
        prefetch(c0 + 1, 1)
        consume(c0, 0)

        @pl.when(c0 + 2 < NCHL)
        def _():
            prefetch(c0 + 2, 0)

        consume(c0 + 1, 1)
        return 0

    lax.fori_loop(0, NCHL // 2, step, 0)
    plsc.subcore_barrier()
    pltpu.sync_copy(z_sp.at[stripe], zcat_hbm.at[gstripe])


@functools.cache
def _make_sc_edge():
    buf = lambda: [pltpu.VMEM((EK,), jnp.int32),
                   pltpu.VMEM((EK,), jnp.int32),
                   pltpu.VMEM((EK, HH), jnp.float32),
                   pltpu.VMEM((EK, HH), jnp.float32),
                   pltpu.VMEM((EK, HH), jnp.float32),
                   pltpu.VMEM((EK // 8, HH), jnp.float32),
                   pltpu.SemaphoreType.DMA]
    return functools.partial(
        pl.kernel,
        out_type=jax.ShapeDtypeStruct((NC, EPE // 8, HH), jnp.float32),
        mesh=_mesh(),
        scratch_types=buf() + buf() + [pltpu.VMEM((HH,), jnp.float32)],
        compiler_params=_SC_PARAMS,
    )(_sc_edge_body)


def _sc_edge_body(tcat_hbm, esrc2_hbm, edst2_hbm, ch_hbm, w2h_hbm, out_hbm,
                  s0, d0, a0, b0, c0, o0, sem0,
                  s1, d1, a1, b1, c1, o1, sem1, w2_v):
    cid = lax.axis_index("c")
    sid = lax.axis_index("s")
    bufs = ((s0, d0, a0, b0, c0, o0, sem0), (s1, d1, a1, b1, c1, o1, sem1))
    pltpu.sync_copy(w2h_hbm.at[cid], w2_v)
    w2regs = [w2_v[pl.ds(k * 16, 16)] for k in range(HH // 16)]

    def off_of(c):
        return pl.multiple_of(sid * ETE + c * EK, 8)

    def off8_of(c):
        return pl.multiple_of((sid * ETE + c * EK) // 8, 8)

    def prefetch(c, b):
        src_v, dst_v, a_v, b_v, c_v, _, sem = bufs[b]
        off = off_of(c)
        pltpu.sync_copy(esrc2_hbm.at[cid, pl.ds(off, EK)], src_v)
        pltpu.sync_copy(edst2_hbm.at[cid, pl.ds(off, EK)], dst_v)
        pltpu.async_copy(tcat_hbm.at[src_v], a_v, sem)
        pltpu.async_copy(tcat_hbm.at[dst_v], b_v, sem)
        pltpu.async_copy(ch_hbm.at[cid, pl.ds(off_of(c), EK)], c_v, sem)

    def consume(c, b):
        src_v, dst_v, a_v, b_v, c_v, o_v, sem = bufs[b]
        pltpu.make_async_copy(tcat_hbm.at[src_v], a_v, sem).wait()
        pltpu.make_async_copy(tcat_hbm.at[dst_v], b_v, sem).wait()
        pltpu.make_async_copy(
            ch_hbm.at[cid, pl.ds(off_of(c), EK)], c_v, sem).wait()

        def row8(r, _):
            for er in range(8):
                e = r * 8 + er
                acc = jnp.zeros((16,), jnp.float32)
                for k in range(HH // 16):
                    sl = pl.ds(k * 16, 16)
                    v = a_v[e, sl] + b_v[e, sl] + c_v[e, sl]
                    acc = acc + jnp.maximum(v, 0.0) * w2regs[k]
                o_v[r, pl.ds(er * 16, 16)] = acc
            return 0

        lax.fori_loop(0, EK // 8, row8, 0)
        pltpu.sync_copy(o_v, out_hbm.at[cid, pl.ds(off8_of(c), EK // 8)])

    prefetch(0, 0)

    def step(c2, _):
        ch = c2 * 2
        prefetch(ch + 1, 1)
        consume(ch, 0)

        @pl.when(ch + 2 < NCHE)
        def _():
            prefetch(ch + 2, 0)

        consume(ch + 1, 1)
        return 0

    lax.fori_loop(0, NCHE // 2, step, 0)


def _run_deg(dstp, ones, zeros):
    return _make_sc_deg()(dstp, ones, zeros)


def _run_layer(ycat, src2, dst):
    return _make_sc_layer()(ycat, src2, dst)


def _run_edge(ab, esrc, edst, cmat, w2v):
    return _make_sc_edge()(ab, esrc, edst, cmat, w2v)


def _tc1_body(x_ref, w_ref, p_ref, y_ref, dinv_ref):
    deg = 1.0 + jnp.sum(p_ref[...], axis=(0, 2))
    dinv = lax.rsqrt(deg).reshape(RB, 1)
    xw = _dot(x_ref[...], w_ref[...])
    y_ref[...] = xw * dinv
    dinv_ref[...] = dinv


def _tc1(x_pad, W1, partials):
    return pl.pallas_call(
        _tc1_body,
        grid=(2, NBLK),
        in_specs=[
            pl.BlockSpec((RB, D), lambda j, i: (i, 0)),
            pl.BlockSpec((D, HH), lambda j, i: (0, j)),
            pl.BlockSpec((NC, RB, 16), lambda j, i: (0, i, 0)),
        ],
        out_specs=[
            pl.BlockSpec((RB, HH), lambda j, i: (j * NBLK + i, 0)),
            pl.BlockSpec((RB, 1), lambda j, i: (i, 0)),
        ],
        out_shape=[
            jax.ShapeDtypeStruct((2 * NP, HH), jnp.float32),
            jax.ShapeDtypeStruct((NP, 1), jnp.float32),
        ],
    )(x_pad, W1, partials)


def _tc2_body(za_ref, zb_ref, dinv_ref, b_ref, w_ref, y_ref):
    dinv = dinv_ref[...]
    z = jnp.concatenate([za_ref[...], zb_ref[...]], axis=1)
    h = jnp.maximum(dinv * z + b_ref[...], 0.0)
    y_ref[...] = _dot(h, w_ref[...]) * dinv


def _tc2(zcat, dinv, b1r, W2):
    return pl.pallas_call(
        _tc2_body,
        grid=(2, NBLK),
        in_specs=[
            pl.BlockSpec((RB, HH), lambda j, i: (i, 0)),
            pl.BlockSpec((RB, HH), lambda j, i: (NBLK + i, 0)),
            pl.BlockSpec((RB, 1), lambda j, i: (i, 0)),
            pl.BlockSpec((1, D), lambda j, i: (0, 0)),
            pl.BlockSpec((D, HH), lambda j, i: (0, j)),
        ],
        out_specs=pl.BlockSpec((RB, HH), lambda j, i: (j * NBLK + i, 0)),
        out_shape=jax.ShapeDtypeStruct((2 * NP, HH), jnp.float32),
    )(zcat, zcat, dinv, b1r, W2)


def _tc3_body(za_ref, zb_ref, dinv_ref, b_ref, we_ref, wn1_ref, bn1_ref,
              wn2_ref, bn2_ref, t_ref, h_ref, n_ref):
    dinv = dinv_ref[...]
    z = jnp.concatenate([za_ref[...], zb_ref[...]], axis=1)
    h = jnp.maximum(dinv * z + b_ref[...], 0.0)
    h_ref[...] = h
    t_ref[...] = _dot(h, we_ref[0])
    nmid = jnp.maximum(_dot(h, wn1_ref[...]) + bn1_ref[...], 0.0)
    n_ref[...] = _dot(nmid, wn2_ref[...]) + bn2_ref[...]


def _tc3(zcat, dinv, b2r, We1ab, Wn1, bn1r, Wn2, bn2r):
    return pl.pallas_call(
        _tc3_body,
        grid=(2, 2, NBLK),
        in_specs=[
            pl.BlockSpec((RB, HH), lambda j, p, i: (i, 0)),
            pl.BlockSpec((RB, HH), lambda j, p, i: (NBLK + i, 0)),
            pl.BlockSpec((RB, 1), lambda j, p, i: (i, 0)),
            pl.BlockSpec((1, D), lambda j, p, i: (0, 0)),
            pl.BlockSpec((1, D, HH), lambda j, p, i: (j, 0, p)),
            pl.BlockSpec((D, HH), lambda j, p, i: (0, 0)),
            pl.BlockSpec((1, HH), lambda j, p, i: (0, 0)),
            pl.BlockSpec((HH, 1), lambda j, p, i: (0, 0)),
            pl.BlockSpec((1, 1), lambda j, p, i: (0, 0)),
        ],
        out_specs=[
            pl.BlockSpec((RB, HH), lambda j, p, i: ((p * 2 + j) * NBLK + i, 0)),
            pl.BlockSpec((RB, D), lambda j, p, i: (i, 0)),
            pl.BlockSpec((RB, 1), lambda j, p, i: (i, 0)),
        ],
        out_shape=[
            jax.ShapeDtypeStruct((4 * NP, HH), jnp.float32),
            jax.ShapeDtypeStruct((NP, D), jnp.float32),
            jax.ShapeDtypeStruct((NP, 1), jnp.float32),
        ],
    )(zcat, zcat, dinv, b2r, We1ab, Wn1, bn1r, Wn2, bn2r)


def _tc3b_body(ea_ref, we_ref, be_ref, c_ref):
    c_ref[0] = _dot(ea_ref[...], we_ref[...]) + be_ref[...]


def _tc3b(ea_pad, We1e, be1r):
    eb = 2048
    return pl.pallas_call(
        _tc3b_body,
        grid=(2, EPE // eb),
        in_specs=[
            pl.BlockSpec((eb, 16), lambda j, i: (i, 0)),
            pl.BlockSpec((16, HH), lambda j, i: (0, j)),
            pl.BlockSpec((1, HH), lambda j, i: (0, j)),
        ],
        out_specs=pl.BlockSpec((1, eb, HH), lambda j, i: (j, i, 0)),
        out_shape=jax.ShapeDtypeStruct((NC, EPE, HH), jnp.float32),
    )(ea_pad, We1e, be1r)


def _tc4_body(ep_ref, be2_ref, out_ref):
    v = ep_ref[0] + ep_ref[1]
    r = lax.broadcasted_iota(jnp.int32, (HH, 8), 0) // 16
    c = lax.broadcasted_iota(jnp.int32, (HH, 8), 1)
    mask = (r == c).astype(jnp.float32)
    out_ref[...] = _dot(v, mask, jax.lax.Precision.HIGHEST) + be2_ref[...]


def _tc4(epart8, be2r):
    eb8 = 512
    return pl.pallas_call(
        _tc4_body,
        grid=(EPE // 8 // eb8,),
        in_specs=[
            pl.BlockSpec((NC, eb8, HH), lambda i: (0, i, 0)),
            pl.BlockSpec((1, 1), lambda i: (0, 0)),
        ],
        out_specs=pl.BlockSpec((eb8, 8), lambda i: (i, 0)),
        out_shape=jax.ShapeDtypeStruct((EPE // 8, 8), jnp.float32),
    )(epart8, be2r)


def kernel(x, edge_index, edge_attr, W1, b1, W2, b2, We1, be1, We2, be2,
           Wn1, bn1, Wn2, bn2):
    src = edge_index[0]
    dst = edge_index[1]

    dstp = jnp.concatenate(
        [dst, jnp.full((EPD - E,), NP - 1, jnp.int32)])
    deg_ones = jnp.full((ETD, 16), 0.0625, jnp.float32)
    deg_zeros = jnp.zeros((NP, 16), jnp.float32)
    pad_idx = (jnp.arange(EPE - E, dtype=jnp.int32) % N)
    srcp = jnp.concatenate([src, pad_idx])
    dstl = jnp.concatenate(
        [dst, N + (jnp.arange(EPE - E, dtype=jnp.int32) % (NP - N))])
    src2 = jnp.stack([srcp, srcp + NP])
    dstp_e = jnp.concatenate([dst, pad_idx])
    esrc2 = jnp.stack([srcp, srcp + 2 * NP])
    edst2 = jnp.stack([dstp_e + NP, dstp_e + 3 * NP])
    x_pad = jnp.pad(x, ((0, NP - N), (0, 0)))
    ea_pad = jnp.pad(edge_attr, ((0, EPE - E), (0, 0)))

    b1r = b1.reshape(1, D)
    b2r = b2.reshape(1, D)
    be1r = be1.reshape(1, D)
    be2r = be2.reshape(1, 1)
    bn1r = bn1.reshape(1, HH)
    bn2r = bn2.reshape(1, 1)
    We1ab = jnp.stack([We1[:D], We1[D:2 * D]])
    We1e = We1[2 * D:]
    w2h = We2[:, 0].reshape(NC, HH)

    partials = _run_deg(dstp, deg_ones, deg_zeros)
    ch = _tc3b(ea_pad, We1e, be1r)
    ycat, dinv = _tc1(x_pad, W1, partials)
    zcat = _run_layer(ycat, src2, dstl)
    y2cat = _tc2(zcat, dinv, b1r, W2)
    z2cat = _run_layer(y2cat, src2, dstl)
    tcat, h_pad, nout = _tc3(z2cat, dinv, b2r, We1ab, Wn1, bn1r, Wn2, bn2r)
    epart8 = _run_edge(tcat, esrc2, edst2, ch, w2h)
    eout = _tc4(epart8, be2r)

    return (eout.reshape(EPE)[:E], nout[:N, 0], h_pad[:N])


# speedup vs baseline: 7.4371x; 1.0284x over previous
import functools

import jax
import jax.numpy as jnp
from jax import lax
from jax.experimental import pallas as pl
from jax.experimental.pallas import tpu as pltpu
from jax.experimental.pallas import tpu_sc as plsc

N = 10000
E = 160000
D = 256
HH = 128
NP = 10240
NBLK = 8
RB = NP // NBLK
NC = 2
NS = 16
RPT = NP // NS

EPD = E + 256
ETD = EPD // 32

EPE = 163840
ETL = EPE // NS
LK = 128
NCHL = ETL // LK

ETE = EPE // NS
EK = 128
NCHE = ETE // EK

def _dot(a, b, prec=jax.lax.Precision.DEFAULT):
    return jnp.dot(a, b, preferred_element_type=jnp.float32, precision=prec)


def _mesh():
    return plsc.VectorSubcoreMesh(
        core_axis_name="c", subcore_axis_name="s",
        num_cores=NC, num_subcores=NS)


_SC_PARAMS = pltpu.CompilerParams(use_tc_tiling_on_sc=False)


@functools.cache
def _make_sc_deg():
    return functools.partial(
        pl.kernel,
        out_type=jax.ShapeDtypeStruct((NC, NP, 16), jnp.float32),
        mesh=_mesh(),
        scratch_types=[
            pltpu.VMEM((ETD,), jnp.int32),
            pltpu.VMEM((ETD, 16), jnp.float32),
            pltpu.VMEM_SHARED((NP, 16), jnp.float32),
        ],
        compiler_params=_SC_PARAMS,
    )(_sc_deg_body)


def _sc_deg_body(dstp_hbm, ones_hbm, zeros_hbm, out_hbm, idx_v, ones_v, deg_sp):
    cid = lax.axis_index("c")
    sid = lax.axis_index("s")
    w = sid * NC + cid
    stripe = pl.ds(pl.multiple_of(sid * RPT, 8), RPT)
    pltpu.sync_copy(zeros_hbm.at[stripe], deg_sp.at[stripe])
    pltpu.sync_copy(ones_hbm, ones_v)
    pltpu.sync_copy(dstp_hbm.at[pl.ds(pl.multiple_of(w * ETD, 8), ETD)], idx_v)
    plsc.subcore_barrier()
    pltpu.sync_copy(ones_v, deg_sp.at[idx_v], add=True)
    plsc.subcore_barrier()
    pltpu.sync_copy(deg_sp.at[stripe], out_hbm.at[cid, stripe])


@functools.cache
def _make_sc_layer():
    buf = lambda: [pltpu.VMEM((LK,), jnp.int32),
                   pltpu.VMEM((LK,), jnp.int32),
                   pltpu.VMEM((LK, HH), jnp.float32),
                   pltpu.SemaphoreType.DMA]
    return functools.partial(
        pl.kernel,
        out_type=jax.ShapeDtypeStruct((2 * NP, HH), jnp.float32),
        mesh=_mesh(),
        scratch_types=buf() + buf() + [
            pltpu.VMEM_SHARED((NP, HH), jnp.float32),
        ],
        compiler_params=_SC_PARAMS,
    )(_sc_layer_body)


def _sc_layer_body(ycat_hbm, src2_hbm, dst_hbm, zcat_hbm,
                   src0, dst0, rows0, sem0, src1, dst1, rows1, sem1, z_sp):
    cid = lax.axis_index("c")
    sid = lax.axis_index("s")
    bufs = ((src0, dst0, rows0, sem0), (src1, dst1, rows1, sem1))
    stripe = pl.ds(pl.multiple_of(sid * RPT, 8), RPT)
    gstripe = pl.ds(pl.multiple_of(cid * NP + sid * RPT, 8), RPT)
    pltpu.sync_copy(ycat_hbm.at[gstripe], z_sp.at[stripe])
    plsc.subcore_barrier()

    def off_of(c):
        return pl.multiple_of(sid * ETL + c * LK, 8)

    def prefetch(c, b):
        src_v, _, rows_v, sem = bufs[b]
        pltpu.sync_copy(src2_hbm.at[cid, pl.ds(off_of(c), LK)], src_v)
        pltpu.async_copy(ycat_hbm.at[src_v], rows_v, sem)

    def consume(c, b):
        src_v, dst_v, rows_v, sem = bufs[b]
        pltpu.sync_copy(dst_hbm.at[pl.ds(off_of(c), LK)], dst_v)
        pltpu.make_async_copy(ycat_hbm.at[src_v], rows_v, sem).wait()
        pltpu.sync_copy(rows_v, z_sp.at[dst_v], add=True)

    prefetch(0, 0)

    def step(c2, _):
        c0 = c2 * 2
        prefetch(c0 + 1, 1)
        consume(c0, 0)

        @pl.when(c0 + 2 < NCHL)
        def _():
            prefetch(c0 + 2, 0)

        consume(c0 + 1, 1)
        return 0

    lax.fori_loop(0, NCHL // 2, step, 0)
    plsc.subcore_barrier()
    pltpu.sync_copy(z_sp.at[stripe], zcat_hbm.at[gstripe])


@functools.cache
def _make_sc_edge():
    buf = lambda: [pltpu.VMEM((EK,), jnp.int32),
                   pltpu.VMEM((EK,), jnp.int32),
                   pltpu.VMEM((EK, HH), jnp.float32),
                   pltpu.VMEM((EK, HH), jnp.float32),
                   pltpu.VMEM((EK, HH), jnp.float32),
                   pltpu.VMEM((EK // 8, HH), jnp.float32),
                   pltpu.SemaphoreType.DMA]
    return functools.partial(
        pl.kernel,
        out_type=jax.ShapeDtypeStruct((NC, EPE // 8, HH), jnp.float32),
        mesh=_mesh(),
        scratch_types=buf() + buf() + [pltpu.VMEM((HH,), jnp.float32)],
        compiler_params=_SC_PARAMS,
    )(_sc_edge_body)


def _sc_edge_body(tcat_hbm, esrc2_hbm, edst2_hbm, ch_hbm, w2h_hbm, out_hbm,
                  s0, d0, a0, b0, c0, o0, sem0,
                  s1, d1, a1, b1, c1, o1, sem1, w2_v):
    cid = lax.axis_index("c")
    sid = lax.axis_index("s")
    bufs = ((s0, d0, a0, b0, c0, o0, sem0), (s1, d1, a1, b1, c1, o1, sem1))
    pltpu.sync_copy(w2h_hbm.at[cid], w2_v)
    w2regs = [w2_v[pl.ds(k * 16, 16)] for k in range(HH // 16)]

    def off_of(c):
        return pl.multiple_of(sid * ETE + c * EK, 8)

    def off8_of(c):
        return pl.multiple_of((sid * ETE + c * EK) // 8, 8)

    def prefetch(c, b):
        src_v, dst_v, a_v, b_v, c_v, _, sem = bufs[b]
        off = off_of(c)
        pltpu.sync_copy(esrc2_hbm.at[cid, pl.ds(off, EK)], src_v)
        pltpu.sync_copy(edst2_hbm.at[cid, pl.ds(off, EK)], dst_v)
        pltpu.async_copy(tcat_hbm.at[src_v], a_v, sem)
        pltpu.async_copy(tcat_hbm.at[dst_v], b_v, sem)
        pltpu.async_copy(ch_hbm.at[cid, pl.ds(off_of(c), EK)], c_v, sem)

    def consume(c, b):
        src_v, dst_v, a_v, b_v, c_v, o_v, sem = bufs[b]
        pltpu.make_async_copy(tcat_hbm.at[src_v], a_v, sem).wait()
        pltpu.make_async_copy(tcat_hbm.at[dst_v], b_v, sem).wait()
        pltpu.make_async_copy(
            ch_hbm.at[cid, pl.ds(off_of(c), EK)], c_v, sem).wait()

        def row8(r, _):
            for er in range(8):
                e = r * 8 + er
                acc = jnp.zeros((16,), jnp.float32)
                for k in range(HH // 16):
                    sl = pl.ds(k * 16, 16)
                    v = a_v[e, sl] + b_v[e, sl] + c_v[e, sl]
                    acc = acc + jnp.maximum(v, 0.0) * w2regs[k]
                o_v[r, pl.ds(er * 16, 16)] = acc
            return 0

        lax.fori_loop(0, EK // 8, row8, 0)
        pltpu.sync_copy(o_v, out_hbm.at[cid, pl.ds(off8_of(c), EK // 8)])

    prefetch(0, 0)

    def step(c2, _):
        ch = c2 * 2
        prefetch(ch + 1, 1)
        consume(ch, 0)

        @pl.when(ch + 2 < NCHE)
        def _():
            prefetch(ch + 2, 0)

        consume(ch + 1, 1)
        return 0

    lax.fori_loop(0, NCHE // 2, step, 0)


def _run_deg(dstp, ones, zeros):
    return _make_sc_deg()(dstp, ones, zeros)


def _run_layer(ycat, src2, dst):
    return _make_sc_layer()(ycat, src2, dst)


def _run_edge(ab, esrc, edst, cmat, w2v):
    return _make_sc_edge()(ab, esrc, edst, cmat, w2v)


def _tc1_body(x_ref, w_ref, p_ref, y_ref, dinv_ref):
    deg = 1.0 + jnp.sum(p_ref[...], axis=(0, 2))
    dinv = lax.rsqrt(deg).reshape(RB, 1)
    xw = _dot(x_ref[...], w_ref[...])
    y_ref[...] = xw * dinv
    dinv_ref[...] = dinv


def _tc1(x_pad, W1, partials):
    return pl.pallas_call(
        _tc1_body,
        grid=(2, NBLK),
        in_specs=[
            pl.BlockSpec((RB, D), lambda j, i: (i, 0)),
            pl.BlockSpec((D, HH), lambda j, i: (0, j)),
            pl.BlockSpec((NC, RB, 16), lambda j, i: (0, i, 0)),
        ],
        out_specs=[
            pl.BlockSpec((RB, HH), lambda j, i: (j * NBLK + i, 0)),
            pl.BlockSpec((RB, 1), lambda j, i: (i, 0)),
        ],
        out_shape=[
            jax.ShapeDtypeStruct((2 * NP, HH), jnp.float32),
            jax.ShapeDtypeStruct((NP, 1), jnp.float32),
        ],
    )(x_pad, W1, partials)


def _tc2_body(za_ref, zb_ref, dinv_ref, b_ref, w_ref, y_ref):
    dinv = dinv_ref[...]
    z = jnp.concatenate([za_ref[...], zb_ref[...]], axis=1)
    h = jnp.maximum(dinv * z + b_ref[...], 0.0)
    y_ref[...] = _dot(h, w_ref[...]) * dinv


def _tc2(zcat, dinv, b1r, W2):
    return pl.pallas_call(
        _tc2_body,
        grid=(2, NBLK),
        in_specs=[
            pl.BlockSpec((RB, HH), lambda j, i: (i, 0)),
            pl.BlockSpec((RB, HH), lambda j, i: (NBLK + i, 0)),
            pl.BlockSpec((RB, 1), lambda j, i: (i, 0)),
            pl.BlockSpec((1, D), lambda j, i: (0, 0)),
            pl.BlockSpec((D, HH), lambda j, i: (0, j)),
        ],
        out_specs=pl.BlockSpec((RB, HH), lambda j, i: (j * NBLK + i, 0)),
        out_shape=jax.ShapeDtypeStruct((2 * NP, HH), jnp.float32),
    )(zcat, zcat, dinv, b1r, W2)


def _tc3_body(za_ref, zb_ref, dinv_ref, b_ref, we_ref, wn1_ref, bn1_ref,
              wn2_ref, bn2_ref, t_ref, h_ref, n_ref):
    dinv = dinv_ref[...]
    z = jnp.concatenate([za_ref[...], zb_ref[...]], axis=1)
    h = jnp.maximum(dinv * z + b_ref[...], 0.0)
    h_ref[...] = h
    t_ref[...] = _dot(h, we_ref[0])
    nmid = jnp.maximum(_dot(h, wn1_ref[...]) + bn1_ref[...], 0.0)
    n_ref[...] = _dot(nmid, wn2_ref[...]) + bn2_ref[...]


def _tc3(zcat, dinv, b2r, We1ab, Wn1, bn1r, Wn2, bn2r):
    return pl.pallas_call(
        _tc3_body,
        grid=(2, 2, NBLK),
        in_specs=[
            pl.BlockSpec((RB, HH), lambda j, p, i: (i, 0)),
            pl.BlockSpec((RB, HH), lambda j, p, i: (NBLK + i, 0)),
            pl.BlockSpec((RB, 1), lambda j, p, i: (i, 0)),
            pl.BlockSpec((1, D), lambda j, p, i: (0, 0)),
            pl.BlockSpec((1, D, HH), lambda j, p, i: (j, 0, p)),
            pl.BlockSpec((D, HH), lambda j, p, i: (0, 0)),
            pl.BlockSpec((1, HH), lambda j, p, i: (0, 0)),
            pl.BlockSpec((HH, 1), lambda j, p, i: (0, 0)),
            pl.BlockSpec((1, 1), lambda j, p, i: (0, 0)),
        ],
        out_specs=[
            pl.BlockSpec((RB, HH), lambda j, p, i: ((p * 2 + j) * NBLK + i, 0)),
            pl.BlockSpec((RB, D), lambda j, p, i: (i, 0)),
            pl.BlockSpec((RB, 1), lambda j, p, i: (i, 0)),
        ],
        out_shape=[
            jax.ShapeDtypeStruct((4 * NP, HH), jnp.float32),
            jax.ShapeDtypeStruct((NP, D), jnp.float32),
            jax.ShapeDtypeStruct((NP, 1), jnp.float32),
        ],
    )(zcat, zcat, dinv, b2r, We1ab, Wn1, bn1r, Wn2, bn2r)


def _tc3b_body(ea_ref, we_ref, be_ref, c_ref):
    c_ref[0] = _dot(ea_ref[...], we_ref[...]) + be_ref[...]


def _tc3b(ea, We1e, be1r):
    eb = 2000
    return pl.pallas_call(
        _tc3b_body,
        grid=(2, E // eb),
        in_specs=[
            pl.BlockSpec((eb, 16), lambda j, i: (i, 0)),
            pl.BlockSpec((16, HH), lambda j, i: (0, j)),
            pl.BlockSpec((1, HH), lambda j, i: (0, j)),
        ],
        out_specs=pl.BlockSpec((1, eb, HH), lambda j, i: (j, i, 0)),
        out_shape=jax.ShapeDtypeStruct((NC, EPE, HH), jnp.float32),
    )(ea, We1e, be1r)


def _tc4_body(ep_ref, be2_ref, out_ref):
    v = ep_ref[0] + ep_ref[1]
    r = lax.broadcasted_iota(jnp.int32, (HH, 8), 0) // 16
    c = lax.broadcasted_iota(jnp.int32, (HH, 8), 1)
    mask = (r == c).astype(jnp.float32)
    out_ref[...] = _dot(v, mask, jax.lax.Precision.HIGHEST) + be2_ref[...]


def _tc4(epart8, be2r):
    eb8 = 512
    return pl.pallas_call(
        _tc4_body,
        grid=(EPE // 8 // eb8,),
        in_specs=[
            pl.BlockSpec((NC, eb8, HH), lambda i: (0, i, 0)),
            pl.BlockSpec((1, 1), lambda i: (0, 0)),
        ],
        out_specs=pl.BlockSpec((eb8, 8), lambda i: (i, 0)),
        out_shape=jax.ShapeDtypeStruct((EPE // 8, 8), jnp.float32),
    )(epart8, be2r)


def kernel(x, edge_index, edge_attr, W1, b1, W2, b2, We1, be1, We2, be2,
           Wn1, bn1, Wn2, bn2):
    src = edge_index[0]
    dst = edge_index[1]

    dstp = jnp.concatenate(
        [dst, jnp.full((EPD - E,), NP - 1, jnp.int32)])
    deg_ones = jnp.full((ETD, 16), 0.0625, jnp.float32)
    deg_zeros = jnp.zeros((NP, 16), jnp.float32)
    pad_idx = (jnp.arange(EPE - E, dtype=jnp.int32) % N)
    srcp = jnp.concatenate([src, pad_idx])
    dstl = jnp.concatenate(
        [dst, N + (jnp.arange(EPE - E, dtype=jnp.int32) % (NP - N))])
    src2 = jnp.stack([srcp, srcp + NP])
    dstp_e = jnp.concatenate([dst, pad_idx])
    esrc2 = jnp.stack([srcp, srcp + 2 * NP])
    edst2 = jnp.stack([dstp_e + NP, dstp_e + 3 * NP])
    x_pad = jnp.pad(x, ((0, NP - N), (0, 0)))

    b1r = b1.reshape(1, D)
    b2r = b2.reshape(1, D)
    be1r = be1.reshape(1, D)
    be2r = be2.reshape(1, 1)
    bn1r = bn1.reshape(1, HH)
    bn2r = bn2.reshape(1, 1)
    We1ab = jnp.stack([We1[:D], We1[D:2 * D]])
    We1e = We1[2 * D:]
    w2h = We2[:, 0].reshape(NC, HH)

    partials = _run_deg(dstp, deg_ones, deg_zeros)
    ch = _tc3b(edge_attr, We1e, be1r)
    ycat, dinv = _tc1(x_pad, W1, partials)
    zcat = _run_layer(ycat, src2, dstl)
    y2cat = _tc2(zcat, dinv, b1r, W2)
    z2cat = _run_layer(y2cat, src2, dstl)
    tcat, h_pad, nout = _tc3(z2cat, dinv, b2r, We1ab, Wn1, bn1r, Wn2, bn2r)
    epart8 = _run_edge(tcat, esrc2, edst2, ch, w2h)
    eout = _tc4(epart8, be2r)

    return (eout.reshape(EPE)[:E], nout[:N, 0], h_pad[:N])
